# baseline (speedup 1.0000x reference)
import functools

import jax
import jax.numpy as jnp
from jax import lax
from jax.experimental import pallas as pl
from jax.experimental.pallas import tpu as pltpu
from jax.experimental.pallas import tpu_sc as plsc

N = 2048
K = 16
L = 16
CHUNKS = N // L
ROWS_PER_W = 256
INF = float("inf")


def _splat_lane(v, lidx):
    dnums = lax.GatherDimensionNumbers(
        offset_dims=(), collapsed_slice_dims=(0,), start_index_map=(0,)
    )
    return lax.gather(
        v, lidx[:, None], dnums, (1,),
        mode=lax.GatherScatterMode.PROMISE_IN_BOUNDS,
    )


def _any(mask):
    return plsc.all_reduce_population_count(mask)[0] > 0


def _lex_less(da, ia, db, ib):
    return (da < db) | ((da == db) & (ia < ib))


def _merge_sorted(ad, ai, bd, bi):
    rd = lax.rev(bd, (0,))
    ri = lax.rev(bi, (0,))
    take_a = _lex_less(ad, ai, rd, ri)
    md = jnp.where(take_a, ad, rd)
    mi = jnp.where(take_a, ai, ri)
    return plsc.sort_key_val(md, mi)


def _merge16(best_d, best_i, cand_d, cand_i):
    cd, ci = plsc.sort_key_val(cand_d, cand_i)
    return _merge_sorted(best_d, best_i, cd, ci)


def _knn_sc_body(pts_hbm, out_hbm, pts_v, out_v, dbuf, cand_d, cand_i):
    wid = lax.axis_index("s") * 2 + lax.axis_index("c")
    b = wid // 8
    i0 = (wid % 8) * ROWS_PER_W
    pltpu.sync_copy(pts_hbm.at[b], pts_v)

    lane = lax.iota(jnp.int32, L)
    inf_vec = jnp.full((L,), INF)
    last_lane = jnp.full((L,), L - 1)
    zero_i = jnp.zeros((L,), jnp.int32)

    def row_body(k, t_prev):
        arow = jnp.minimum(k, ROWS_PER_W - 1)
        a_off = (k % 2) * N
        b_off = ((k + 1) % 2) * N
        brow = jnp.maximum(k - 1, 0)

        i = i0 + arow
        qbase = (i // L) * L
        lidx = jnp.full((L,), i % L)
        xi = _splat_lane(pts_v[0, pl.ds(qbase, L)], lidx)
        yi = _splat_lane(pts_v[1, pl.ds(qbase, L)], lidx)
        zi = _splat_lane(pts_v[2, pl.ds(qbase, L)], lidx)

        fused_init = ((inf_vec, inf_vec), jnp.full((L,), -1, jnp.int32))

        @plsc.parallel_loop(0, CHUNKS, 1, unroll=16, carry=fused_init)
        def fused(c, carry):
            (m1, m2), pm1c = carry
            base = c * L
            xj = pts_v[0, pl.ds(base, L)]
            yj = pts_v[1, pl.ds(base, L)]
            zj = pts_v[2, pl.ds(base, L)]
            dx = xj - xi
            dy = yj - yi
            dz = zj - zi
            d = (dx * dx + dy * dy) + dz * dz
            dbuf[pl.ds(a_off + base, L)] = d
            m2n = jnp.minimum(m2, jnp.maximum(m1, d))
            m1n = jnp.minimum(m1, d)
            db = dbuf[pl.ds(b_off + base, L)]
            m = db <= t_prev
            pos = plsc.cumsum(m.astype(jnp.int32))
            idxs = pm1c + pos
            plsc.store_scatter(cand_d, [idxs], db, mask=m)
            plsc.store_scatter(cand_i, [idxs], lane + base, mask=m)
            pm1n = pm1c + plsc.all_reduce_population_count(m)
            return ((m1n, m2n), pm1n)

        (_m1, m2), pm1 = fused

        d_self = dbuf[pl.ds(a_off + qbase, L)]
        dbuf[pl.ds(a_off + qbase, L)] = jnp.where(lane == lidx, INF, d_self)

        sm2, _sv = plsc.sort_key_val(m2, lane)
        t_next = _splat_lane(sm2, last_lane)

        n_cand = pm1[0] + 1
        n_cand_vec = jnp.full((L,), n_cand)

        def load_c(j):
            base = j * L
            d = cand_d[pl.ds(base, L)]
            ix = cand_i[pl.ds(base, L)]
            d = jnp.where(lane + base < n_cand_vec, d, INF)
            return d, ix

        s0d, s0i = load_c(0)
        s1d, s1i = load_c(1)
        q0d, q0i = plsc.sort_key_val(s0d, s0i)
        q1d, q1i = plsc.sort_key_val(s1d, s1i)
        best_d, best_i = _merge_sorted(q0d, q0i, q1d, q1i)

        def chunk_c(j, carry):
            bd, bi = carry
            d, ix = load_c(j)
            nd, ni = _merge16(bd, bi, d, ix)
            return (nd, ni)

        best_d, best_i = lax.fori_loop(
            2, (n_cand + L - 1) // L, chunk_c, (best_d, best_i)
        )
        out_v[brow, :] = best_i
        return t_next

    neg_inf_vec = jnp.full((L,), -INF)
    lax.fori_loop(0, ROWS_PER_W + 1, row_body, neg_inf_vec)
    pltpu.sync_copy(out_v, out_hbm.at[b, pl.ds(i0, ROWS_PER_W)])


@jax.jit
def kernel(features, points):
    del features
    b, n, _ = points.shape
    pts_t = jnp.transpose(points, (0, 2, 1))
    mesh = plsc.VectorSubcoreMesh(core_axis_name="c", subcore_axis_name="s")
    topk = pl.kernel(
        _knn_sc_body,
        out_type=jax.ShapeDtypeStruct((b, n, K), jnp.int32),
        mesh=mesh,
        scratch_types=[
            pltpu.VMEM((3, N), jnp.float32),
            pltpu.VMEM((ROWS_PER_W, K), jnp.int32),
            pltpu.VMEM((2 * N,), jnp.float32),
            pltpu.VMEM((N + L,), jnp.float32),
            pltpu.VMEM((N + L,), jnp.int32),
        ],
        compiler_params=pltpu.CompilerParams(needs_layout_passes=False),
    )(pts_t)
    batch_ids = jnp.broadcast_to(
        jnp.arange(b, dtype=jnp.int32).reshape(b, 1, 1, 1), (b, n, K, 1)
    )
    return jnp.concatenate([batch_ids, topk[..., None]], axis=3)


# speedup vs baseline: 1.0799x; 1.0799x over previous
import functools

import jax
import jax.numpy as jnp
from jax import lax
from jax.experimental import pallas as pl
from jax.experimental.pallas import tpu as pltpu
from jax.experimental.pallas import tpu_sc as plsc

N = 2048
K = 16
L = 16
CHUNKS = N // L
ROWS_PER_W = 256
INF = float("inf")


def _splat_lane(v, lidx):
    dnums = lax.GatherDimensionNumbers(
        offset_dims=(), collapsed_slice_dims=(0,), start_index_map=(0,)
    )
    return lax.gather(
        v, lidx[:, None], dnums, (1,),
        mode=lax.GatherScatterMode.PROMISE_IN_BOUNDS,
    )


def _any(mask):
    return plsc.all_reduce_population_count(mask)[0] > 0


def _lex_less(da, ia, db, ib):
    return (da < db) | ((da == db) & (ia < ib))


def _merge_sorted(ad, ai, bd, bi):
    rd = lax.rev(bd, (0,))
    ri = lax.rev(bi, (0,))
    take_a = _lex_less(ad, ai, rd, ri)
    md = jnp.where(take_a, ad, rd)
    mi = jnp.where(take_a, ai, ri)
    return plsc.sort_key_val(md, mi)


def _merge16(best_d, best_i, cand_d, cand_i):
    cd, ci = plsc.sort_key_val(cand_d, cand_i)
    return _merge_sorted(best_d, best_i, cd, ci)


def _knn_sc_body(pts_hbm, out_hbm, pts_v, out_v, dbuf, cand_d, cand_i):
    wid = lax.axis_index("s") * 2 + lax.axis_index("c")
    b = wid // 8
    i0 = (wid % 8) * ROWS_PER_W
    pltpu.sync_copy(pts_hbm.at[b], pts_v)

    lane = lax.iota(jnp.int32, L)
    inf_vec = jnp.full((L,), INF)
    last_lane = jnp.full((L,), L - 1)
    zero_i = jnp.zeros((L,), jnp.int32)

    def row_body(k, t_prev):
        arow = jnp.minimum(k, ROWS_PER_W - 1)
        a_off = (k % 2) * N
        b_off = ((k + 1) % 2) * N
        brow = jnp.maximum(k - 1, 0)

        i = i0 + arow
        qbase = (i // L) * L
        lidx = jnp.full((L,), i % L)
        xi = _splat_lane(pts_v[0, pl.ds(qbase, L)], lidx)
        yi = _splat_lane(pts_v[1, pl.ds(qbase, L)], lidx)
        zi = _splat_lane(pts_v[2, pl.ds(qbase, L)], lidx)

        fused_init = ((inf_vec, inf_vec), jnp.full((L,), -1, jnp.int32))

        @plsc.parallel_loop(0, CHUNKS, 1, unroll=8, carry=fused_init)
        def fused(c, carry):
            (m1, m2), pm1c = carry
            base = c * L
            xj = pts_v[0, pl.ds(base, L)]
            yj = pts_v[1, pl.ds(base, L)]
            zj = pts_v[2, pl.ds(base, L)]
            dx = xj - xi
            dy = yj - yi
            dz = zj - zi
            d = (dx * dx + dy * dy) + dz * dz
            dbuf[pl.ds(a_off + base, L)] = d
            m2n = jnp.minimum(m2, jnp.maximum(m1, d))
            m1n = jnp.minimum(m1, d)
            db = dbuf[pl.ds(b_off + base, L)]
            m = db <= t_prev
            pos = plsc.cumsum(m.astype(jnp.int32))
            idxs = pm1c + pos
            plsc.store_scatter(cand_d, [idxs], db, mask=m)
            plsc.store_scatter(cand_i, [idxs], lane + base, mask=m)
            pm1n = pm1c + plsc.all_reduce_population_count(m)
            return ((m1n, m2n), pm1n)

        (_m1, m2), pm1 = fused

        d_self = dbuf[pl.ds(a_off + qbase, L)]
        dbuf[pl.ds(a_off + qbase, L)] = jnp.where(lane == lidx, INF, d_self)

        sm2, _sv = plsc.sort_key_val(m2, lane)
        t_next = _splat_lane(sm2, last_lane)

        n_cand = pm1[0] + 1
        n_cand_vec = jnp.full((L,), n_cand)

        def load_c(j):
            base = j * L
            d = cand_d[pl.ds(base, L)]
            ix = cand_i[pl.ds(base, L)]
            d = jnp.where(lane + base < n_cand_vec, d, INF)
            return d, ix

        s0d, s0i = load_c(0)
        s1d, s1i = load_c(1)
        q0d, q0i = plsc.sort_key_val(s0d, s0i)
        q1d, q1i = plsc.sort_key_val(s1d, s1i)
        best_d, best_i = _merge_sorted(q0d, q0i, q1d, q1i)

        def chunk_c(j, carry):
            bd, bi = carry
            d, ix = load_c(j)
            nd, ni = _merge16(bd, bi, d, ix)
            return (nd, ni)

        best_d, best_i = lax.fori_loop(
            2, (n_cand + L - 1) // L, chunk_c, (best_d, best_i)
        )
        out_v[brow, :] = best_i
        return t_next

    neg_inf_vec = jnp.full((L,), -INF)
    lax.fori_loop(0, ROWS_PER_W + 1, row_body, neg_inf_vec)
    pltpu.sync_copy(out_v, out_hbm.at[b, pl.ds(i0, ROWS_PER_W)])


@jax.jit
def kernel(features, points):
    del features
    b, n, _ = points.shape
    pts_t = jnp.transpose(points, (0, 2, 1))
    mesh = plsc.VectorSubcoreMesh(core_axis_name="c", subcore_axis_name="s")
    topk = pl.kernel(
        _knn_sc_body,
        out_type=jax.ShapeDtypeStruct((b, n, K), jnp.int32),
        mesh=mesh,
        scratch_types=[
            pltpu.VMEM((3, N), jnp.float32),
            pltpu.VMEM((ROWS_PER_W, K), jnp.int32),
            pltpu.VMEM((2 * N,), jnp.float32),
            pltpu.VMEM((N + L,), jnp.float32),
            pltpu.VMEM((N + L,), jnp.int32),
        ],
        compiler_params=pltpu.CompilerParams(needs_layout_passes=False),
    )(pts_t)
    batch_ids = jnp.broadcast_to(
        jnp.arange(b, dtype=jnp.int32).reshape(b, 1, 1, 1), (b, n, K, 1)
    )
    return jnp.concatenate([batch_ids, topk[..., None]], axis=3)


# speedup vs baseline: 1.1791x; 1.0919x over previous
import functools

import jax
import jax.numpy as jnp
from jax import lax
from jax.experimental import pallas as pl
from jax.experimental.pallas import tpu as pltpu
from jax.experimental.pallas import tpu_sc as plsc

N = 2048
K = 16
L = 16
CHUNKS = N // L
ROWS_PER_W = 256
INF = float("inf")


def _splat_lane(v, lidx):
    dnums = lax.GatherDimensionNumbers(
        offset_dims=(), collapsed_slice_dims=(0,), start_index_map=(0,)
    )
    return lax.gather(
        v, lidx[:, None], dnums, (1,),
        mode=lax.GatherScatterMode.PROMISE_IN_BOUNDS,
    )


def _any(mask):
    return plsc.all_reduce_population_count(mask)[0] > 0


def _lex_less(da, ia, db, ib):
    return (da < db) | ((da == db) & (ia < ib))


def _merge_sorted(ad, ai, bd, bi):
    rd = lax.rev(bd, (0,))
    ri = lax.rev(bi, (0,))
    take_a = _lex_less(ad, ai, rd, ri)
    md = jnp.where(take_a, ad, rd)
    mi = jnp.where(take_a, ai, ri)
    return plsc.sort_key_val(md, mi)


def _merge16(best_d, best_i, cand_d, cand_i):
    cd, ci = plsc.sort_key_val(cand_d, cand_i)
    return _merge_sorted(best_d, best_i, cd, ci)


def _knn_sc_body(pts_hbm, out_hbm, pts_v, out_v, dbuf, cand_d, cand_i):
    wid = lax.axis_index("s") * 2 + lax.axis_index("c")
    b = wid // 8
    i0 = (wid % 8) * ROWS_PER_W
    pltpu.sync_copy(pts_hbm.at[b], pts_v)

    lane = lax.iota(jnp.int32, L)
    inf_vec = jnp.full((L,), INF)
    t_lane = jnp.full((L,), 8)
    zero_i = jnp.zeros((L,), jnp.int32)

    def row_body(k, t_prev):
        arow = jnp.minimum(k, ROWS_PER_W - 1)
        a_off = (k % 2) * N
        b_off = ((k + 1) % 2) * N
        brow = jnp.maximum(k - 1, 0)

        i = i0 + arow
        qbase = (i // L) * L
        lidx = jnp.full((L,), i % L)
        xi = _splat_lane(pts_v[0, pl.ds(qbase, L)], lidx)
        yi = _splat_lane(pts_v[1, pl.ds(qbase, L)], lidx)
        zi = _splat_lane(pts_v[2, pl.ds(qbase, L)], lidx)

        fused_init = ((inf_vec, inf_vec), jnp.full((L,), -1, jnp.int32))

        @plsc.parallel_loop(0, CHUNKS, 1, unroll=8, carry=fused_init)
        def fused(c, carry):
            (m1, m2), pm1c = carry
            base = c * L
            xj = pts_v[0, pl.ds(base, L)]
            yj = pts_v[1, pl.ds(base, L)]
            zj = pts_v[2, pl.ds(base, L)]
            dx = xj - xi
            dy = yj - yi
            dz = zj - zi
            d = (dx * dx + dy * dy) + dz * dz
            dbuf[pl.ds(a_off + base, L)] = d
            m2n = jnp.minimum(m2, jnp.maximum(m1, d))
            m1n = jnp.minimum(m1, d)
            db = dbuf[pl.ds(b_off + base, L)]
            m = db <= t_prev
            pos = plsc.cumsum(m.astype(jnp.int32))
            idxs = pm1c + pos
            plsc.store_scatter(cand_d, [idxs], db, mask=m)
            plsc.store_scatter(cand_i, [idxs], lane + base, mask=m)
            pm1n = pm1c + plsc.all_reduce_population_count(m)
            return ((m1n, m2n), pm1n)

        (_m1, m2), pm1 = fused

        d_self = dbuf[pl.ds(a_off + qbase, L)]
        dbuf[pl.ds(a_off + qbase, L)] = jnp.where(lane == lidx, INF, d_self)

        sm2, _sv = plsc.sort_key_val(m2, lane)
        t_next = _splat_lane(sm2, t_lane)

        n_cand = pm1[0] + 1
        n_cand_vec = jnp.full((L,), n_cand)

        def load_c(j):
            base = j * L
            d = cand_d[pl.ds(base, L)]
            ix = cand_i[pl.ds(base, L)]
            d = jnp.where(lane + base < n_cand_vec, d, INF)
            return d, ix

        s0d, s0i = load_c(0)
        s1d, s1i = load_c(1)
        q0d, q0i = plsc.sort_key_val(s0d, s0i)
        q1d, q1i = plsc.sort_key_val(s1d, s1i)
        best_d, best_i = _merge_sorted(q0d, q0i, q1d, q1i)

        def chunk_c(j, carry):
            bd, bi = carry
            d, ix = load_c(j)
            nd, ni = _merge16(bd, bi, d, ix)
            return (nd, ni)

        best_d, best_i = lax.fori_loop(
            2, (n_cand + L - 1) // L, chunk_c, (best_d, best_i)
        )
        out_v[brow, :] = best_i
        return t_next

    neg_inf_vec = jnp.full((L,), -INF)
    lax.fori_loop(0, ROWS_PER_W + 1, row_body, neg_inf_vec)
    pltpu.sync_copy(out_v, out_hbm.at[b, pl.ds(i0, ROWS_PER_W)])


@jax.jit
def kernel(features, points):
    del features
    b, n, _ = points.shape
    pts_t = jnp.transpose(points, (0, 2, 1))
    mesh = plsc.VectorSubcoreMesh(core_axis_name="c", subcore_axis_name="s")
    topk = pl.kernel(
        _knn_sc_body,
        out_type=jax.ShapeDtypeStruct((b, n, K), jnp.int32),
        mesh=mesh,
        scratch_types=[
            pltpu.VMEM((3, N), jnp.float32),
            pltpu.VMEM((ROWS_PER_W, K), jnp.int32),
            pltpu.VMEM((2 * N,), jnp.float32),
            pltpu.VMEM((N + L,), jnp.float32),
            pltpu.VMEM((N + L,), jnp.int32),
        ],
        compiler_params=pltpu.CompilerParams(needs_layout_passes=False),
    )(pts_t)
    batch_ids = jnp.broadcast_to(
        jnp.arange(b, dtype=jnp.int32).reshape(b, 1, 1, 1), (b, n, K, 1)
    )
    return jnp.concatenate([batch_ids, topk[..., None]], axis=3)
